# baseline (speedup 1.0000x reference)
import jax
import jax.numpy as jnp
from jax import lax
from jax.experimental import pallas as pl
from jax.experimental.pallas import tpu as pltpu
from jax.experimental.pallas import tpu_sc as plsc

N = 16384
D = 64
DA = 80
NNZ = 268435
NW = 32
E = 128
CPT = 66
NNZ_PAD = NW * E * CPT
RPT = N // 16
GRPT = N // NW
CB = 128
NSYNC = 5

_mesh = plsc.VectorSubcoreMesh(core_axis_name="c", subcore_axis_name="s")


def _scale_chunk(gb, vals_v, j):
    @plsc.parallel_loop(0, E, step=16, unroll=2)
    def _(e0):
        vv = vals_v[j, pl.ds(e0, 16)]
        for g in range(16):
            v = lax.gather(
                vv, jnp.full((16, 1), g, dtype=jnp.int32),
                lax.GatherDimensionNumbers(offset_dims=(),
                                           collapsed_slice_dims=(0,),
                                           start_index_map=(0,)),
                slice_sizes=(1,),
                mode=lax.GatherScatterMode.PROMISE_IN_BOUNDS)
            for q in range(DA // 16):
                sl = (e0 + g, pl.ds(q * 16, 16))
                gb[sl] = gb[sl] * v


def _sc_body(xa_hbm, cols_hbm, rows_hbm, vals_hbm, zero_hbm, flag_hbm,
             z1_hbm, z2_hbm, z3_hbm, pex_hbm,
             cols_v, rows_v, vals_v, gbuf, flagw_v, flagr_v,
             acc, gsem0, gsem1, fsem):
    c = lax.axis_index("c")
    s = lax.axis_index("s")
    w = c * 16 + s

    def cross_sync(p):
        plsc.subcore_barrier()

        @pl.when(s == 0)
        def _():
            flagw_v[...] = jnp.full((16,), 2.0, jnp.float32)
            pltpu.sync_copy(flagw_v, flag_hbm.at[c, p])

        @pl.loop(0, 256, init_carry=jnp.bool_(False))
        def _poll(i, done):
            @pl.when(jnp.logical_not(done))
            def _():
                pltpu.async_copy(flag_hbm.at[1 - c, p], flagr_v, fsem).wait()
            return jnp.logical_or(done, flagr_v[...][0] >= 1.0)

        plsc.subcore_barrier()

    def sparse_pass(src_hbm, dst_hbm, k):
        pltpu.async_copy(src_hbm.at[cols_v.at[0]], gbuf.at[0], gsem0)
        pltpu.async_copy(src_hbm.at[cols_v.at[1]], gbuf.at[1], gsem1)

        @pl.loop(0, CPT, step=2)
        def _(j):
            pltpu.make_async_copy(src_hbm.at[cols_v.at[j]], gbuf.at[0],
                                  gsem0).wait()
            _scale_chunk(gbuf.at[0], vals_v, j)
            pltpu.sync_copy(gbuf.at[0], acc.at[rows_v.at[j]], add=True)

            @pl.when(j + 2 < CPT)
            def _():
                pltpu.async_copy(src_hbm.at[cols_v.at[j + 2]], gbuf.at[0],
                                 gsem0)

            pltpu.make_async_copy(src_hbm.at[cols_v.at[j + 1]], gbuf.at[1],
                                  gsem1).wait()
            _scale_chunk(gbuf.at[1], vals_v, j + 1)
            pltpu.sync_copy(gbuf.at[1], acc.at[rows_v.at[j + 1]], add=True)

            @pl.when(j + 3 < CPT)
            def _():
                pltpu.async_copy(src_hbm.at[cols_v.at[j + 3]], gbuf.at[1],
                                 gsem1)

        plsc.subcore_barrier()
        pltpu.sync_copy(acc.at[pl.ds(s * RPT, RPT)],
                        pex_hbm.at[c, pl.ds(s * RPT, RPT)])
        cross_sync(2 * k)
        for half in range(GRPT // CB):
            base = w * GRPT + half * CB
            pltpu.sync_copy(pex_hbm.at[0, pl.ds(base, CB)], gbuf.at[0])
            pltpu.sync_copy(pex_hbm.at[1, pl.ds(base, CB)], gbuf.at[1])

            @plsc.parallel_loop(0, CB, step=1, unroll=2)
            def _(r):
                for q in range(DA // 16):
                    sl = (r, pl.ds(q * 16, 16))
                    gbuf[(0, r) + sl[1:]] = gbuf[(0, r) + sl[1:]] + \
                        gbuf[(1, r) + sl[1:]]

            pltpu.sync_copy(gbuf.at[0], dst_hbm.at[pl.ds(base, CB)])
        if k < 2:
            pltpu.sync_copy(zero_hbm, acc.at[pl.ds(s * RPT, RPT)])
            cross_sync(2 * k + 1)

    pltpu.sync_copy(cols_hbm.at[w], cols_v)
    pltpu.sync_copy(rows_hbm.at[w], rows_v)
    pltpu.sync_copy(vals_hbm.at[w], vals_v)
    pltpu.sync_copy(zero_hbm, acc.at[pl.ds(s * RPT, RPT)])
    plsc.subcore_barrier()

    sparse_pass(xa_hbm, z1_hbm, 0)
    sparse_pass(z1_hbm, z2_hbm, 1)
    sparse_pass(z2_hbm, z3_hbm, 2)


_sc_chain = pl.kernel(
    _sc_body,
    out_type=(jax.ShapeDtypeStruct((N, DA), jnp.float32),
              jax.ShapeDtypeStruct((N, DA), jnp.float32),
              jax.ShapeDtypeStruct((N, DA), jnp.float32),
              jax.ShapeDtypeStruct((2, N, DA), jnp.float32)),
    mesh=_mesh,
    scratch_types=[
        pltpu.VMEM((CPT, E), jnp.int32),
        pltpu.VMEM((CPT, E), jnp.int32),
        pltpu.VMEM((CPT, E), jnp.float32),
        pltpu.VMEM((2, E, DA), jnp.float32),
        pltpu.VMEM((16,), jnp.float32),
        pltpu.VMEM((16,), jnp.float32),
        pltpu.VMEM_SHARED((N, DA), jnp.float32),
        pltpu.SemaphoreType.DMA,
        pltpu.SemaphoreType.DMA,
        pltpu.SemaphoreType.DMA,
    ],
    compiler_params=pltpu.CompilerParams(use_tc_tiling_on_sc=False),
)


def _tc_fin_body(x0_ref, z1_ref, z2_ref, z3_ref, w0_ref, w1_ref, w2_ref,
                 b0_ref, b1_ref, b2_ref, out_ref):
    cd = (((1,), (1,)), ((), ()))

    def matT(x, w_ref):
        return lax.dot_general(x, w_ref[...], cd,
                               preferred_element_type=jnp.float32)

    z1 = z1_ref[:, :D]
    z2 = z2_ref[:, :D]
    z3 = z3_ref[:, :D]
    s = z1_ref[:, D:D + 1]
    t1 = z2_ref[:, D:D + 1]
    t2 = z3_ref[:, D:D + 1]
    b0 = b0_ref[...]
    b1 = b1_ref[...]
    b2 = b2_ref[...]

    acc = x0_ref[...] + matT(z1, w0_ref)
    acc = acc + matT(matT(z2, w0_ref), w1_ref)
    acc = acc + matT(matT(matT(z3, w0_ref), w1_ref), w2_ref)
    b0w1 = matT(b0, w1_ref)
    acc = acc + s * (b0 + b1 + b2)
    acc = acc + t1 * (b0w1 + matT(b1, w2_ref))
    acc = acc + t2 * matT(b0w1, w2_ref)
    out_ref[...] = acc


_TCB = 2048

_tc_fin = pl.pallas_call(
    _tc_fin_body,
    grid=(N // _TCB,),
    in_specs=[
        pl.BlockSpec((_TCB, D), lambda i: (i, 0)),
        pl.BlockSpec((_TCB, DA), lambda i: (i, 0)),
        pl.BlockSpec((_TCB, DA), lambda i: (i, 0)),
        pl.BlockSpec((_TCB, DA), lambda i: (i, 0)),
        pl.BlockSpec((D, D), lambda i: (0, 0)),
        pl.BlockSpec((D, D), lambda i: (0, 0)),
        pl.BlockSpec((D, D), lambda i: (0, 0)),
        pl.BlockSpec((1, D), lambda i: (0, 0)),
        pl.BlockSpec((1, D), lambda i: (0, 0)),
        pl.BlockSpec((1, D), lambda i: (0, 0)),
    ],
    out_specs=pl.BlockSpec((_TCB, D), lambda i: (i, 0)),
    out_shape=jax.ShapeDtypeStruct((N, D), jnp.float32),
)


def kernel(edge_embedding, rows, cols, vals, W0, b0, W1, b1, W2, b2):
    pad = NNZ_PAD - NNZ
    ar = jnp.arange(pad, dtype=jnp.int32) % N
    cols_p = jnp.concatenate([cols, ar]).reshape(NW, CPT, E)
    rows_p = jnp.concatenate([rows, ar]).reshape(NW, CPT, E)
    vals_p = jnp.concatenate(
        [vals, jnp.zeros((pad,), jnp.float32)]).reshape(NW, CPT, E)
    xa = jnp.concatenate(
        [edge_embedding,
         jnp.ones((N, 1), jnp.float32),
         jnp.zeros((N, DA - D - 1), jnp.float32)], axis=1)
    zblock = jnp.zeros((RPT, DA), jnp.float32)
    flag = (jnp.clip(vals[:2 * NSYNC * 16], -1.0, 1.0) * 1e-30).reshape(
        2, NSYNC, 16)

    z1, z2, z3, _ = _sc_chain(xa, cols_p, rows_p, vals_p, zblock, flag)
    return _tc_fin(edge_embedding, z1, z2, z3, W0, W1, W2,
                   b0.reshape(1, D), b1.reshape(1, D), b2.reshape(1, D))
